# baseline (speedup 1.0000x reference)
import functools

import jax
import jax.numpy as jnp
from jax.experimental import pallas as pl

_NEG_INF = float("-inf")
_BIG_I32 = 2 ** 30
_TOTAL_TOKENS = 59


def _seg_topk_core(x, *, k):
    r_blk, s_n, l_n = x.shape
    mseg = jnp.max(x, axis=2, keepdims=True)
    m0 = jnp.max(mseg, axis=1, keepdims=True)
    sumexp = jnp.sum(jnp.sum(jnp.exp(x - jnp.broadcast_to(m0, x.shape)),
                             axis=2, keepdims=True), axis=1, keepdims=True)
    lse = m0 + jnp.log(sumexp)

    eye = (jax.lax.broadcasted_iota(jnp.int32, (r_blk, s_n, s_n), 1) ==
           jax.lax.broadcasted_iota(jnp.int32, (r_blk, s_n, s_n), 2))
    eye = eye.astype(jnp.float32)
    f = jnp.broadcast_to(mseg, (r_blk, s_n, s_n))
    g = jnp.sum(f * eye, axis=1, keepdims=True)
    gb = jnp.broadcast_to(g, (r_blk, s_n, s_n))
    i1 = jax.lax.broadcasted_iota(jnp.int32, (r_blk, s_n, s_n), 1)
    i2 = jax.lax.broadcasted_iota(jnp.int32, (r_blk, s_n, s_n), 2)
    beats = (gb > f) | ((gb == f) & (i2 < i1))
    rankseg = jnp.sum(beats.astype(jnp.float32), axis=2, keepdims=True)
    sel = (rankseg < k).astype(jnp.float32)
    sel_g = jnp.sum(jnp.broadcast_to(sel, (r_blk, s_n, s_n)) * eye,
                    axis=1, keepdims=True)
    pos = jnp.sum(jnp.broadcast_to(sel_g, (r_blk, s_n, s_n)) *
                  (i2 < i1).astype(jnp.float32), axis=2, keepdims=True)
    pos_g = jnp.sum(jnp.broadcast_to(pos, (r_blk, s_n, s_n)) * eye,
                    axis=1, keepdims=True)
    ttk = jax.lax.broadcasted_iota(jnp.int32, (r_blk, k, s_n), 1)
    ttk = ttk.astype(jnp.float32)
    oht = ((jnp.broadcast_to(pos_g, (r_blk, k, s_n)) == ttk) &
           (jnp.broadcast_to(sel_g, (r_blk, k, s_n)) > 0)).astype(jnp.float32)
    segid = jnp.sum(
        oht * jax.lax.broadcasted_iota(jnp.int32, (r_blk, k, s_n), 2)
        .astype(jnp.float32), axis=2, keepdims=True)

    comp_rows = []
    for r in range(r_blk):
        oh_r = jnp.reshape(jax.lax.slice(oht, (r, 0, 0), (r + 1, k, s_n)),
                           (k, s_n))
        x_r = jnp.reshape(jax.lax.slice(x, (r, 0, 0), (r + 1, s_n, l_n)),
                          (s_n, l_n))
        c_r = jax.lax.dot_general(
            oh_r, x_r, (((1,), (0,)), ((), ())),
            precision=jax.lax.Precision.HIGHEST,
            preferred_element_type=jnp.float32)
        comp_rows.append(jnp.reshape(c_r, (1, k, l_n)))
    c = jnp.concatenate(comp_rows, axis=0)

    fc = (jax.lax.broadcasted_iota(jnp.int32, (r_blk, k, l_n), 1) * l_n +
          jax.lax.broadcasted_iota(jnp.int32, (r_blk, k, l_n), 2))
    it_k = jax.lax.broadcasted_iota(jnp.int32, (r_blk, k, 1), 1)
    it_k = it_k.astype(jnp.float32)
    vals = []
    gidx = []
    for j in range(k):
        m = jnp.max(jnp.max(c, axis=2, keepdims=True), axis=1, keepdims=True)
        cand = jnp.where(c == jnp.broadcast_to(m, c.shape), fc, _BIG_I32)
        idx = jnp.min(jnp.min(cand, axis=2, keepdims=True), axis=1,
                      keepdims=True)
        tj = jnp.zeros_like(m)
        for u in range(1, k):
            tj = tj + (idx >= u * l_n).astype(jnp.float32)
        posj = idx.astype(jnp.float32) - tj * l_n
        sid = jnp.sum(segid * (it_k == jnp.broadcast_to(tj, it_k.shape))
                      .astype(jnp.float32), axis=1, keepdims=True)
        vals.append(m)
        gidx.append(sid * l_n + posj)
        if j < k - 1:
            c = jnp.where(fc == jnp.broadcast_to(idx, fc.shape), _NEG_INF, c)
    return (jnp.concatenate(vals, axis=2), jnp.concatenate(gidx, axis=2), lse)


def _seg_topk_kernel(x_ref, topv_ref, topi_ref, lse_ref, *, k):
    tv, tif, lse = _seg_topk_core(x_ref[...], k=k)
    topv_ref[...] = tv
    topi_ref[...] = tif.astype(jnp.int32)
    lse_ref[...] = lse


def _eye_f32(n):
    r = jax.lax.broadcasted_iota(jnp.int32, (n, n), 0)
    c = jax.lax.broadcasted_iota(jnp.int32, (n, n), 1)
    return (r == c).astype(jnp.float32)


def _row2col(row):
    n = row.shape[1]
    b = jnp.broadcast_to(row, (n, n))
    return jnp.sum(_eye_f32(n) * b, axis=1, keepdims=True)


def _div10(x_i32):
    return jax.lax.shift_right_logical(x_i32 * 6554, 16)


def _rank_row_to_sel(s_row, n):
    s_col_b = jnp.broadcast_to(_row2col(s_row), (n, n))
    s_row_b = jnp.broadcast_to(s_row, (n, n))
    ii = jax.lax.broadcasted_iota(jnp.int32, (n, n), 0)
    jj = jax.lax.broadcasted_iota(jnp.int32, (n, n), 1)
    gt = (s_row_b > s_col_b) | ((s_row_b == s_col_b) & (jj < ii))
    return jnp.sum(gt.astype(jnp.float32), axis=1, keepdims=True)


def _topk_row(s_row, n, k):
    rank_col = _rank_row_to_sel(s_row, n)
    tt = jax.lax.broadcasted_iota(jnp.int32, (n, k), 1).astype(jnp.float32)
    z = (jnp.broadcast_to(rank_col, (n, k)) == tt).astype(jnp.float32)
    s_col_b = jnp.broadcast_to(_row2col(s_row), (n, k))
    vals = jnp.sum(z * s_col_b, axis=0, keepdims=True)
    ii = jax.lax.broadcasted_iota(jnp.int32, (n, k), 0).astype(jnp.float32)
    idxs = jnp.sum(z * ii, axis=0, keepdims=True)
    return vals, idxs


def _flatten_to_row(mat, r, c):
    n = r * c
    g = jnp.concatenate([mat] * r, axis=1)
    ffr = jax.lax.broadcasted_iota(jnp.int32, (r, n), 1)
    ppr = jax.lax.broadcasted_iota(jnp.int32, (r, n), 0)
    fdiv = _div10(ffr) if c == 10 else ffr // c
    amask = (ppr == fdiv).astype(jnp.float32)
    return jnp.sum(g * amask, axis=0, keepdims=True)


def _tree_kernel(xlast_ref, tv2_ref, ti2_ref, ls2_ref, tok_ref,
                 dt_ref, tsp_ref, tm_ref, tpi_ref, *, k, d, total):
    nrows = 1 + d * k
    nbranch = d * k
    nflat = nrows * k
    tv1, ti1f, ls1 = _seg_topk_core(xlast_ref[...], k=k)
    lp1 = jnp.reshape(tv1 - jnp.broadcast_to(ls1, tv1.shape), (1, k))
    tv2 = jax.lax.slice(jnp.reshape(tv2_ref[...], (tv2_ref.shape[0], k)),
                        (0, 0), (nbranch, k))
    ls2 = jax.lax.slice(jnp.reshape(ls2_ref[...], (ls2_ref.shape[0], 1)),
                        (0, 0), (nbranch, 1))
    lp2 = tv2 - jnp.broadcast_to(ls2, tv2.shape)
    lp = jnp.concatenate([lp1, lp2], axis=0)
    ti2 = jax.lax.slice(jnp.reshape(ti2_ref[...], (ti2_ref.shape[0], k)),
                        (0, 0), (nbranch, k))
    topi_f = jnp.concatenate(
        [jnp.reshape(ti1f, (1, k)), ti2.astype(jnp.float32)], axis=0)

    scores_row = jax.lax.slice(lp, (0, 0), (1, k))
    score_segs = [scores_row]
    parent_segs = [jnp.zeros((1, 1), jnp.float32)]
    for i in range(d):
        nxt = jax.lax.slice(lp, (1 + i * k, 0), (1 + (i + 1) * k, k))
        sc_col_b = jnp.broadcast_to(_row2col(scores_row), (k, k))
        cu = nxt + sc_col_b
        cu_flat = _flatten_to_row(cu, k, k)
        score_segs.append(cu_flat)
        new_scores, idx_row = _topk_row(cu_flat, k * k, k)
        off = 1 + k * k * max(0, i - 1) + (k if i > 0 else 0)
        parent_segs.append(idx_row + jnp.float32(off))
        scores_row = new_scores

    s_flat = jnp.concatenate(score_segs, axis=1)
    ss_flat = _flatten_to_row(topi_f, nrows, k)
    parents_row = jnp.concatenate(parent_segs, axis=1)

    n = nflat
    rank_col = _rank_row_to_sel(s_flat, n)
    sel_col = (rank_col < total).astype(jnp.float32)

    sel_row = jnp.sum(_eye_f32(n) * jnp.broadcast_to(sel_col, (n, n)),
                      axis=0, keepdims=True)
    ii = jax.lax.broadcasted_iota(jnp.int32, (n, n), 0)
    jj = jax.lax.broadcasted_iota(jnp.int32, (n, n), 1)
    p_col = jnp.sum(jnp.broadcast_to(sel_row, (n, n)) * (jj < ii), axis=1,
                    keepdims=True)

    tt = jax.lax.broadcasted_iota(jnp.int32, (n, total), 1).astype(jnp.float32)
    b59 = (jnp.broadcast_to(p_col, (n, total)) == tt).astype(jnp.float32)
    b59 = b59 * jnp.broadcast_to(sel_col, (n, total))
    ii_f = jax.lax.broadcasted_iota(jnp.int32, (n, total), 0).astype(jnp.float32)
    tsi_row = jnp.sum(b59 * ii_f, axis=0, keepdims=True)
    ss_col_b = jnp.broadcast_to(_row2col(ss_flat), (n, total))
    tokens_row = jnp.sum(b59 * ss_col_b, axis=0, keepdims=True)

    z59 = (jnp.broadcast_to(rank_col, (n, total)) == tt).astype(jnp.float32)
    s_col_b = jnp.broadcast_to(_row2col(s_flat), (n, total))
    tsp_ref[...] = jnp.sum(z59 * s_col_b, axis=0, keepdims=True)

    g_row = _div10(tsi_row.astype(jnp.int32))
    gp = jnp.broadcast_to(g_row, (nrows, total))
    pp = jax.lax.broadcasted_iota(jnp.int32, (nrows, total), 0)
    oh = (pp == gp).astype(jnp.float32)
    par_col_b = jnp.broadcast_to(_row2col(parents_row), (nrows, total))
    dp_row = jnp.sum(oh * par_col_b, axis=0, keepdims=True)

    tsi_col_b = jnp.broadcast_to(_row2col(tsi_row), (total, total))
    dp_b = jnp.broadcast_to(dp_row, (total, total))
    cnt = jnp.sum((tsi_col_b < dp_b - 1.0).astype(jnp.float32), axis=0,
                  keepdims=True)
    mask_index = jnp.where(dp_row == 0.0, -1.0, cnt) + 1.0
    mi_row = jnp.clip(mask_index, 0.0, float(total))

    nt = total + 1
    rr = jax.lax.broadcasted_iota(jnp.int32, (nt, nt), 0)
    cc = jax.lax.broadcasted_iota(jnp.int32, (nt, nt), 1)
    mi_col = _row2col(mi_row)
    mfull = jnp.concatenate([jnp.zeros((1, 1), jnp.float32), mi_col], axis=0)
    mb = jnp.broadcast_to(mfull, (nt, nt))
    oh = cc.astype(jnp.float32) == mb
    rr_f = rr.astype(jnp.float32)
    isfwd = (mb > rr_f - 1.0) & (rr >= 1)
    base = ((rr == cc) | (cc == 0) | (oh & isfwd)).astype(jnp.float32)
    adj = (oh & (~isfwd) & (rr >= 1)).astype(jnp.float32)
    reach = jnp.maximum(adj, ((rr == cc)).astype(jnp.float32))
    for _ in range(6):
        sq = jax.lax.dot_general(reach, reach, (((1,), (0,)), ((), ())),
                                 preferred_element_type=jnp.float32)
        reach = (reach + sq > 0.0).astype(jnp.float32)
    fin = jax.lax.dot_general(reach, base, (((1,), (0,)), ((), ())),
                              preferred_element_type=jnp.float32)
    tm = (fin > 0.0).astype(jnp.float32)
    tm_ref[...] = tm

    tpi_ref[...] = (jnp.sum(tm, axis=1, keepdims=True) - 1.0).astype(jnp.int32)
    tok_col = _row2col(tokens_row)
    dt = jnp.concatenate(
        [tok_ref[...].astype(jnp.float32), tok_col], axis=0)
    dt_ref[...] = dt.astype(jnp.int32)


def kernel(last_logits, branch_logits, sample_token, total_tokens, depth, top_k):
    d, k, vocab = branch_logits.shape
    total = _TOTAL_TOKENS
    nrows = 1 + d * k
    nbranch = d * k
    rblk = 16
    npad = ((nbranch + rblk - 1) // rblk) * rblk

    seg = 100
    seglen = vocab // seg
    nbpad = -(-nbranch // rblk) * rblk
    grid_r = nbpad // rblk

    tv2, ti2, ls2 = pl.pallas_call(
        functools.partial(_seg_topk_kernel, k=k),
        grid=(grid_r,),
        in_specs=[pl.BlockSpec((rblk, seg, seglen), lambda i: (i, 0, 0))],
        out_specs=[
            pl.BlockSpec((rblk, 1, k), lambda i: (i, 0, 0)),
            pl.BlockSpec((rblk, 1, k), lambda i: (i, 0, 0)),
            pl.BlockSpec((rblk, 1, 1), lambda i: (i, 0, 0)),
        ],
        out_shape=[
            jax.ShapeDtypeStruct((nbpad, 1, k), jnp.float32),
            jax.ShapeDtypeStruct((nbpad, 1, k), jnp.int32),
            jax.ShapeDtypeStruct((nbpad, 1, 1), jnp.float32),
        ],
    )(branch_logits.reshape(nbranch, seg, seglen))

    nt = total + 1
    dt, tsp, tm, tpi = pl.pallas_call(
        functools.partial(_tree_kernel, k=k, d=d, total=total),
        out_shape=[
            jax.ShapeDtypeStruct((nt, 1), jnp.int32),
            jax.ShapeDtypeStruct((1, total), jnp.float32),
            jax.ShapeDtypeStruct((nt, nt), jnp.float32),
            jax.ShapeDtypeStruct((nt, 1), jnp.int32),
        ],
    )(last_logits.reshape(1, seg, seglen), tv2, ti2, ls2, sample_token)

    return (
        dt.reshape(1, nt),
        tsp.reshape(total),
        tm.reshape(1, 1, nt, nt),
        tpi.reshape(nt),
    )


# speedup vs baseline: 1.0328x; 1.0328x over previous
import functools

import jax
import jax.numpy as jnp
from jax.experimental import pallas as pl

_NEG_INF = float("-inf")
_BIG_I32 = 2 ** 30
_TOTAL_TOKENS = 59


def _seg_topk_core(x, *, k):
    r_blk, s_n, l_n = x.shape
    mseg = jnp.max(x, axis=2, keepdims=True)
    m0 = jnp.max(mseg, axis=1, keepdims=True)
    sumexp = jnp.sum(jnp.sum(jnp.exp(x - jnp.broadcast_to(m0, x.shape)),
                             axis=2, keepdims=True), axis=1, keepdims=True)
    lse = m0 + jnp.log(sumexp)

    eye = (jax.lax.broadcasted_iota(jnp.int32, (r_blk, s_n, s_n), 1) ==
           jax.lax.broadcasted_iota(jnp.int32, (r_blk, s_n, s_n), 2))
    eye = eye.astype(jnp.float32)
    f = jnp.broadcast_to(mseg, (r_blk, s_n, s_n))
    g = jnp.sum(f * eye, axis=1, keepdims=True)
    gb = jnp.broadcast_to(g, (r_blk, s_n, s_n))
    i1 = jax.lax.broadcasted_iota(jnp.int32, (r_blk, s_n, s_n), 1)
    i2 = jax.lax.broadcasted_iota(jnp.int32, (r_blk, s_n, s_n), 2)
    beats = (gb > f) | ((gb == f) & (i2 < i1))
    rankseg = jnp.sum(beats.astype(jnp.float32), axis=2, keepdims=True)
    sel = (rankseg < k).astype(jnp.float32)
    sel_g = jnp.sum(jnp.broadcast_to(sel, (r_blk, s_n, s_n)) * eye,
                    axis=1, keepdims=True)
    pos = jnp.sum(jnp.broadcast_to(sel_g, (r_blk, s_n, s_n)) *
                  (i2 < i1).astype(jnp.float32), axis=2, keepdims=True)
    pos_g = jnp.sum(jnp.broadcast_to(pos, (r_blk, s_n, s_n)) * eye,
                    axis=1, keepdims=True)
    ttk = jax.lax.broadcasted_iota(jnp.int32, (r_blk, k, s_n), 1)
    ttk = ttk.astype(jnp.float32)
    oht = ((jnp.broadcast_to(pos_g, (r_blk, k, s_n)) == ttk) &
           (jnp.broadcast_to(sel_g, (r_blk, k, s_n)) > 0)).astype(jnp.float32)
    segid = jnp.sum(
        oht * jax.lax.broadcasted_iota(jnp.int32, (r_blk, k, s_n), 2)
        .astype(jnp.float32), axis=2, keepdims=True)

    comp_rows = []
    for r in range(r_blk):
        oh_r = jnp.reshape(jax.lax.slice(oht, (r, 0, 0), (r + 1, k, s_n)),
                           (k, s_n))
        x_r = jnp.reshape(jax.lax.slice(x, (r, 0, 0), (r + 1, s_n, l_n)),
                          (s_n, l_n))
        c_r = jax.lax.dot_general(
            oh_r, x_r, (((1,), (0,)), ((), ())),
            precision=jax.lax.Precision.HIGHEST,
            preferred_element_type=jnp.float32)
        comp_rows.append(jnp.reshape(c_r, (1, k, l_n)))
    c = jnp.concatenate(comp_rows, axis=0)

    fc = (jax.lax.broadcasted_iota(jnp.int32, (r_blk, k, l_n), 1) * l_n +
          jax.lax.broadcasted_iota(jnp.int32, (r_blk, k, l_n), 2))
    it_k = jax.lax.broadcasted_iota(jnp.int32, (r_blk, k, 1), 1)
    it_k = it_k.astype(jnp.float32)
    vals = []
    gidx = []
    for j in range(k):
        m = jnp.max(jnp.max(c, axis=2, keepdims=True), axis=1, keepdims=True)
        cand = jnp.where(c == jnp.broadcast_to(m, c.shape), fc, _BIG_I32)
        idx = jnp.min(jnp.min(cand, axis=2, keepdims=True), axis=1,
                      keepdims=True)
        tj = jnp.zeros_like(m)
        for u in range(1, k):
            tj = tj + (idx >= u * l_n).astype(jnp.float32)
        posj = idx.astype(jnp.float32) - tj * l_n
        sid = jnp.sum(segid * (it_k == jnp.broadcast_to(tj, it_k.shape))
                      .astype(jnp.float32), axis=1, keepdims=True)
        vals.append(m)
        gidx.append(sid * l_n + posj)
        if j < k - 1:
            c = jnp.where(fc == jnp.broadcast_to(idx, fc.shape), _NEG_INF, c)
    return (jnp.concatenate(vals, axis=2), jnp.concatenate(gidx, axis=2), lse)


def _seg_topk_kernel(x_ref, topv_ref, topi_ref, lse_ref, *, k):
    tv, tif, lse = _seg_topk_core(x_ref[...], k=k)
    topv_ref[...] = tv
    topi_ref[...] = tif.astype(jnp.int32)
    lse_ref[...] = lse


def _eye_f32(n):
    r = jax.lax.broadcasted_iota(jnp.int32, (n, n), 0)
    c = jax.lax.broadcasted_iota(jnp.int32, (n, n), 1)
    return (r == c).astype(jnp.float32)


def _row2col(row):
    n = row.shape[1]
    b = jnp.broadcast_to(row, (n, n))
    return jnp.sum(_eye_f32(n) * b, axis=1, keepdims=True)


def _div10(x_i32):
    return jax.lax.shift_right_logical(x_i32 * 6554, 16)


def _rank_row_to_sel(s_row, n):
    s_col_b = jnp.broadcast_to(_row2col(s_row), (n, n))
    s_row_b = jnp.broadcast_to(s_row, (n, n))
    ii = jax.lax.broadcasted_iota(jnp.int32, (n, n), 0)
    jj = jax.lax.broadcasted_iota(jnp.int32, (n, n), 1)
    gt = (s_row_b > s_col_b) | ((s_row_b == s_col_b) & (jj < ii))
    return jnp.sum(gt.astype(jnp.float32), axis=1, keepdims=True)


def _topk_row(s_row, n, k):
    rank_col = _rank_row_to_sel(s_row, n)
    tt = jax.lax.broadcasted_iota(jnp.int32, (n, k), 1).astype(jnp.float32)
    z = (jnp.broadcast_to(rank_col, (n, k)) == tt).astype(jnp.float32)
    s_col_b = jnp.broadcast_to(_row2col(s_row), (n, k))
    vals = jnp.sum(z * s_col_b, axis=0, keepdims=True)
    ii = jax.lax.broadcasted_iota(jnp.int32, (n, k), 0).astype(jnp.float32)
    idxs = jnp.sum(z * ii, axis=0, keepdims=True)
    return vals, idxs


def _flatten_to_row(mat, r, c):
    n = r * c
    g = jnp.concatenate([mat] * r, axis=1)
    ffr = jax.lax.broadcasted_iota(jnp.int32, (r, n), 1)
    ppr = jax.lax.broadcasted_iota(jnp.int32, (r, n), 0)
    fdiv = _div10(ffr) if c == 10 else ffr // c
    amask = (ppr == fdiv).astype(jnp.float32)
    return jnp.sum(g * amask, axis=0, keepdims=True)


def _tree_kernel(xlast_ref, tv2_ref, ti2_ref, ls2_ref, tok_ref,
                 dt_ref, tsp_ref, tm_ref, tpi_ref, *, k, d, total):
    nrows = 1 + d * k
    nbranch = d * k
    nflat = nrows * k
    tv1, ti1f, ls1 = _seg_topk_core(xlast_ref[...], k=k)
    lp1 = jnp.reshape(tv1 - jnp.broadcast_to(ls1, tv1.shape), (1, k))
    tv2 = jax.lax.slice(jnp.reshape(tv2_ref[...], (tv2_ref.shape[0], k)),
                        (0, 0), (nbranch, k))
    ls2 = jax.lax.slice(jnp.reshape(ls2_ref[...], (ls2_ref.shape[0], 1)),
                        (0, 0), (nbranch, 1))
    lp2 = tv2 - jnp.broadcast_to(ls2, tv2.shape)
    lp = jnp.concatenate([lp1, lp2], axis=0)
    ti2 = jax.lax.slice(jnp.reshape(ti2_ref[...], (ti2_ref.shape[0], k)),
                        (0, 0), (nbranch, k))
    topi_f = jnp.concatenate(
        [jnp.reshape(ti1f, (1, k)), ti2.astype(jnp.float32)], axis=0)

    scores_row = jax.lax.slice(lp, (0, 0), (1, k))
    score_segs = [scores_row]
    parent_segs = [jnp.zeros((1, 1), jnp.float32)]
    for i in range(d):
        nxt = jax.lax.slice(lp, (1 + i * k, 0), (1 + (i + 1) * k, k))
        sc_col_b = jnp.broadcast_to(_row2col(scores_row), (k, k))
        cu = nxt + sc_col_b
        cu_flat = _flatten_to_row(cu, k, k)
        score_segs.append(cu_flat)
        new_scores, idx_row = _topk_row(cu_flat, k * k, k)
        off = 1 + k * k * max(0, i - 1) + (k if i > 0 else 0)
        parent_segs.append(idx_row + jnp.float32(off))
        scores_row = new_scores

    s_flat = jnp.concatenate(score_segs, axis=1)
    ss_flat = _flatten_to_row(topi_f, nrows, k)
    parents_row = jnp.concatenate(parent_segs, axis=1)

    n = nflat
    rank_col = _rank_row_to_sel(s_flat, n)
    sel_col = (rank_col < total).astype(jnp.float32)

    sel_row = jnp.sum(_eye_f32(n) * jnp.broadcast_to(sel_col, (n, n)),
                      axis=0, keepdims=True)
    ii = jax.lax.broadcasted_iota(jnp.int32, (n, n), 0)
    jj = jax.lax.broadcasted_iota(jnp.int32, (n, n), 1)
    p_col = jnp.sum(jnp.broadcast_to(sel_row, (n, n)) * (jj < ii), axis=1,
                    keepdims=True)

    tt = jax.lax.broadcasted_iota(jnp.int32, (n, total), 1).astype(jnp.float32)
    b59 = (jnp.broadcast_to(p_col, (n, total)) == tt).astype(jnp.float32)
    b59 = b59 * jnp.broadcast_to(sel_col, (n, total))
    ii_f = jax.lax.broadcasted_iota(jnp.int32, (n, total), 0).astype(jnp.float32)
    tsi_row = jnp.sum(b59 * ii_f, axis=0, keepdims=True)
    ss_col_b = jnp.broadcast_to(_row2col(ss_flat), (n, total))
    tokens_row = jnp.sum(b59 * ss_col_b, axis=0, keepdims=True)

    z59 = (jnp.broadcast_to(rank_col, (n, total)) == tt).astype(jnp.float32)
    s_col_b = jnp.broadcast_to(_row2col(s_flat), (n, total))
    tsp_ref[...] = jnp.sum(z59 * s_col_b, axis=0, keepdims=True)

    g_row = _div10(tsi_row.astype(jnp.int32))
    gp = jnp.broadcast_to(g_row, (nrows, total))
    pp = jax.lax.broadcasted_iota(jnp.int32, (nrows, total), 0)
    oh = (pp == gp).astype(jnp.float32)
    par_col_b = jnp.broadcast_to(_row2col(parents_row), (nrows, total))
    dp_row = jnp.sum(oh * par_col_b, axis=0, keepdims=True)

    tsi_col_b = jnp.broadcast_to(_row2col(tsi_row), (total, total))
    dp_b = jnp.broadcast_to(dp_row, (total, total))
    cnt = jnp.sum((tsi_col_b < dp_b - 1.0).astype(jnp.float32), axis=0,
                  keepdims=True)
    mask_index = jnp.where(dp_row == 0.0, -1.0, cnt) + 1.0
    mi_row = jnp.clip(mask_index, 0.0, float(total))

    nt = total + 1
    rr = jax.lax.broadcasted_iota(jnp.int32, (nt, nt), 0)
    cc = jax.lax.broadcasted_iota(jnp.int32, (nt, nt), 1)
    mi_col = _row2col(mi_row)
    mfull = jnp.concatenate([jnp.zeros((1, 1), jnp.float32), mi_col], axis=0)
    mb = jnp.broadcast_to(mfull, (nt, nt))
    oh = cc.astype(jnp.float32) == mb
    rr_f = rr.astype(jnp.float32)
    isfwd = (mb > rr_f - 1.0) & (rr >= 1)
    base = ((rr == cc) | (cc == 0) | (oh & isfwd)).astype(jnp.float32)
    adj = (oh & (~isfwd) & (rr >= 1)).astype(jnp.float32)
    reach = jnp.maximum(adj, ((rr == cc)).astype(jnp.float32))
    for _ in range(6):
        sq = jax.lax.dot_general(reach, reach, (((1,), (0,)), ((), ())),
                                 preferred_element_type=jnp.float32)
        reach = (reach + sq > 0.0).astype(jnp.float32)
    fin = jax.lax.dot_general(reach, base, (((1,), (0,)), ((), ())),
                              preferred_element_type=jnp.float32)
    tm = (fin > 0.0).astype(jnp.float32)
    tm_ref[...] = tm

    tpi_ref[...] = (jnp.sum(tm, axis=1, keepdims=True) - 1.0).astype(jnp.int32)
    tok_col = _row2col(tokens_row)
    dt = jnp.concatenate(
        [tok_ref[...].astype(jnp.float32), tok_col], axis=0)
    dt_ref[...] = dt.astype(jnp.int32)


def kernel(last_logits, branch_logits, sample_token, total_tokens, depth, top_k):
    d, k, vocab = branch_logits.shape
    total = _TOTAL_TOKENS
    nrows = 1 + d * k
    nbranch = d * k
    rblk = 8
    npad = ((nbranch + rblk - 1) // rblk) * rblk

    seg = 100
    seglen = vocab // seg
    nbpad = -(-nbranch // rblk) * rblk
    grid_r = nbpad // rblk

    tv2, ti2, ls2 = pl.pallas_call(
        functools.partial(_seg_topk_kernel, k=k),
        grid=(grid_r,),
        in_specs=[pl.BlockSpec((rblk, seg, seglen), lambda i: (i, 0, 0))],
        out_specs=[
            pl.BlockSpec((rblk, 1, k), lambda i: (i, 0, 0)),
            pl.BlockSpec((rblk, 1, k), lambda i: (i, 0, 0)),
            pl.BlockSpec((rblk, 1, 1), lambda i: (i, 0, 0)),
        ],
        out_shape=[
            jax.ShapeDtypeStruct((nbpad, 1, k), jnp.float32),
            jax.ShapeDtypeStruct((nbpad, 1, k), jnp.int32),
            jax.ShapeDtypeStruct((nbpad, 1, 1), jnp.float32),
        ],
    )(branch_logits.reshape(nbranch, seg, seglen))

    nt = total + 1
    dt, tsp, tm, tpi = pl.pallas_call(
        functools.partial(_tree_kernel, k=k, d=d, total=total),
        out_shape=[
            jax.ShapeDtypeStruct((nt, 1), jnp.int32),
            jax.ShapeDtypeStruct((1, total), jnp.float32),
            jax.ShapeDtypeStruct((nt, nt), jnp.float32),
            jax.ShapeDtypeStruct((nt, 1), jnp.int32),
        ],
    )(last_logits.reshape(1, seg, seglen), tv2, ti2, ls2, sample_token)

    return (
        dt.reshape(1, nt),
        tsp.reshape(total),
        tm.reshape(1, 1, nt, nt),
        tpi.reshape(nt),
    )


# speedup vs baseline: 1.0442x; 1.0110x over previous
import functools

import jax
import jax.numpy as jnp
from jax.experimental import pallas as pl
from jax.experimental.pallas import tpu as pltpu

_NEG_INF = float("-inf")
_BIG_I32 = 2 ** 30
_TOTAL_TOKENS = 59


def _seg_topk_core(x, *, k):
    r_blk, s_n, l_n = x.shape
    mseg = jnp.max(x, axis=2, keepdims=True)
    m0 = jnp.max(mseg, axis=1, keepdims=True)
    sumexp = jnp.sum(jnp.sum(jnp.exp(x - jnp.broadcast_to(m0, x.shape)),
                             axis=2, keepdims=True), axis=1, keepdims=True)
    lse = m0 + jnp.log(sumexp)

    eye = (jax.lax.broadcasted_iota(jnp.int32, (r_blk, s_n, s_n), 1) ==
           jax.lax.broadcasted_iota(jnp.int32, (r_blk, s_n, s_n), 2))
    eye = eye.astype(jnp.float32)
    f = jnp.broadcast_to(mseg, (r_blk, s_n, s_n))
    g = jnp.sum(f * eye, axis=1, keepdims=True)
    gb = jnp.broadcast_to(g, (r_blk, s_n, s_n))
    i1 = jax.lax.broadcasted_iota(jnp.int32, (r_blk, s_n, s_n), 1)
    i2 = jax.lax.broadcasted_iota(jnp.int32, (r_blk, s_n, s_n), 2)
    beats = (gb > f) | ((gb == f) & (i2 < i1))
    rankseg = jnp.sum(beats.astype(jnp.float32), axis=2, keepdims=True)
    sel = (rankseg < k).astype(jnp.float32)
    sel_g = jnp.sum(jnp.broadcast_to(sel, (r_blk, s_n, s_n)) * eye,
                    axis=1, keepdims=True)
    pos = jnp.sum(jnp.broadcast_to(sel_g, (r_blk, s_n, s_n)) *
                  (i2 < i1).astype(jnp.float32), axis=2, keepdims=True)
    pos_g = jnp.sum(jnp.broadcast_to(pos, (r_blk, s_n, s_n)) * eye,
                    axis=1, keepdims=True)
    ttk = jax.lax.broadcasted_iota(jnp.int32, (r_blk, k, s_n), 1)
    ttk = ttk.astype(jnp.float32)
    oht = ((jnp.broadcast_to(pos_g, (r_blk, k, s_n)) == ttk) &
           (jnp.broadcast_to(sel_g, (r_blk, k, s_n)) > 0)).astype(jnp.float32)
    segid = jnp.sum(
        oht * jax.lax.broadcasted_iota(jnp.int32, (r_blk, k, s_n), 2)
        .astype(jnp.float32), axis=2, keepdims=True)

    comp_rows = []
    for r in range(r_blk):
        oh_r = jnp.reshape(jax.lax.slice(oht, (r, 0, 0), (r + 1, k, s_n)),
                           (k, s_n))
        x_r = jnp.reshape(jax.lax.slice(x, (r, 0, 0), (r + 1, s_n, l_n)),
                          (s_n, l_n))
        c_r = jax.lax.dot_general(
            oh_r, x_r, (((1,), (0,)), ((), ())),
            precision=jax.lax.Precision.HIGHEST,
            preferred_element_type=jnp.float32)
        comp_rows.append(jnp.reshape(c_r, (1, k, l_n)))
    c = jnp.concatenate(comp_rows, axis=0)

    fc = (jax.lax.broadcasted_iota(jnp.int32, (r_blk, k, l_n), 1) * l_n +
          jax.lax.broadcasted_iota(jnp.int32, (r_blk, k, l_n), 2))
    it_k = jax.lax.broadcasted_iota(jnp.int32, (r_blk, k, 1), 1)
    it_k = it_k.astype(jnp.float32)
    vals = []
    gidx = []
    for j in range(k):
        m = jnp.max(jnp.max(c, axis=2, keepdims=True), axis=1, keepdims=True)
        cand = jnp.where(c == jnp.broadcast_to(m, c.shape), fc, _BIG_I32)
        idx = jnp.min(jnp.min(cand, axis=2, keepdims=True), axis=1,
                      keepdims=True)
        tj = jnp.zeros_like(m)
        for u in range(1, k):
            tj = tj + (idx >= u * l_n).astype(jnp.float32)
        posj = idx.astype(jnp.float32) - tj * l_n
        sid = jnp.sum(segid * (it_k == jnp.broadcast_to(tj, it_k.shape))
                      .astype(jnp.float32), axis=1, keepdims=True)
        vals.append(m)
        gidx.append(sid * l_n + posj)
        if j < k - 1:
            c = jnp.where(fc == jnp.broadcast_to(idx, fc.shape), _NEG_INF, c)
    return (jnp.concatenate(vals, axis=2), jnp.concatenate(gidx, axis=2), lse)


def _eye_f32(n):
    r = jax.lax.broadcasted_iota(jnp.int32, (n, n), 0)
    c = jax.lax.broadcasted_iota(jnp.int32, (n, n), 1)
    return (r == c).astype(jnp.float32)


def _row2col(row):
    n = row.shape[1]
    b = jnp.broadcast_to(row, (n, n))
    return jnp.sum(_eye_f32(n) * b, axis=1, keepdims=True)


def _div10(x_i32):
    return jax.lax.shift_right_logical(x_i32 * 6554, 16)


def _rank_row_to_sel(s_row, n):
    s_col_b = jnp.broadcast_to(_row2col(s_row), (n, n))
    s_row_b = jnp.broadcast_to(s_row, (n, n))
    ii = jax.lax.broadcasted_iota(jnp.int32, (n, n), 0)
    jj = jax.lax.broadcasted_iota(jnp.int32, (n, n), 1)
    gt = (s_row_b > s_col_b) | ((s_row_b == s_col_b) & (jj < ii))
    return jnp.sum(gt.astype(jnp.float32), axis=1, keepdims=True)


def _topk_row(s_row, n, k):
    rank_col = _rank_row_to_sel(s_row, n)
    tt = jax.lax.broadcasted_iota(jnp.int32, (n, k), 1).astype(jnp.float32)
    z = (jnp.broadcast_to(rank_col, (n, k)) == tt).astype(jnp.float32)
    s_col_b = jnp.broadcast_to(_row2col(s_row), (n, k))
    vals = jnp.sum(z * s_col_b, axis=0, keepdims=True)
    ii = jax.lax.broadcasted_iota(jnp.int32, (n, k), 0).astype(jnp.float32)
    idxs = jnp.sum(z * ii, axis=0, keepdims=True)
    return vals, idxs


def _flatten_to_row(mat, r, c):
    n = r * c
    g = jnp.concatenate([mat] * r, axis=1)
    ffr = jax.lax.broadcasted_iota(jnp.int32, (r, n), 1)
    ppr = jax.lax.broadcasted_iota(jnp.int32, (r, n), 0)
    fdiv = _div10(ffr) if c == 10 else ffr // c
    amask = (ppr == fdiv).astype(jnp.float32)
    return jnp.sum(g * amask, axis=0, keepdims=True)


def _fused_kernel(xb_ref, xl_ref, tok_ref,
                  dt_ref, tsp_ref, tm_ref, tpi_ref,
                  tv_s, ti_s, ls_s, *, k, d, total, nsteps):
    pid = pl.program_id(0)
    rblk = xb_ref.shape[0]

    @pl.when(pid < nsteps)
    def _stage_a():
        tv, tif, lse = _seg_topk_core(xb_ref[...], k=k)
        r0 = pid * rblk
        tv_s[pl.ds(r0, rblk), :] = jnp.reshape(tv, (rblk, k))
        ti_s[pl.ds(r0, rblk), :] = jnp.reshape(tif, (rblk, k))
        ls_s[pl.ds(r0, rblk), :] = jnp.reshape(lse, (rblk, 1))

    @pl.when(pid == nsteps)
    def _stage_b():
        _tree_body(xl_ref, tok_ref, dt_ref, tsp_ref, tm_ref, tpi_ref,
                   tv_s, ti_s, ls_s, k=k, d=d, total=total)


def _tree_body(xlast_ref, tok_ref, dt_ref, tsp_ref, tm_ref, tpi_ref,
               tv_s, ti_s, ls_s, *, k, d, total):
    nrows = 1 + d * k
    nbranch = d * k
    nflat = nrows * k
    tv1, ti1f, ls1 = _seg_topk_core(xlast_ref[...], k=k)
    lp1 = jnp.reshape(tv1 - jnp.broadcast_to(ls1, tv1.shape), (1, k))
    tv2 = jax.lax.slice(tv_s[...], (0, 0), (nbranch, k))
    ls2 = jax.lax.slice(ls_s[...], (0, 0), (nbranch, 1))
    lp2 = tv2 - jnp.broadcast_to(ls2, tv2.shape)
    lp = jnp.concatenate([lp1, lp2], axis=0)
    ti2f = jax.lax.slice(ti_s[...], (0, 0), (nbranch, k))
    topi_f = jnp.concatenate([jnp.reshape(ti1f, (1, k)), ti2f], axis=0)

    scores_row = jax.lax.slice(lp, (0, 0), (1, k))
    score_segs = [scores_row]
    parent_segs = [jnp.zeros((1, 1), jnp.float32)]
    for i in range(d):
        nxt = jax.lax.slice(lp, (1 + i * k, 0), (1 + (i + 1) * k, k))
        sc_col_b = jnp.broadcast_to(_row2col(scores_row), (k, k))
        cu = nxt + sc_col_b
        cu_flat = _flatten_to_row(cu, k, k)
        score_segs.append(cu_flat)
        new_scores, idx_row = _topk_row(cu_flat, k * k, k)
        off = 1 + k * k * max(0, i - 1) + (k if i > 0 else 0)
        parent_segs.append(idx_row + jnp.float32(off))
        scores_row = new_scores

    s_flat = jnp.concatenate(score_segs, axis=1)
    ss_flat = _flatten_to_row(topi_f, nrows, k)
    parents_row = jnp.concatenate(parent_segs, axis=1)

    n = nflat
    rank_col = _rank_row_to_sel(s_flat, n)
    sel_col = (rank_col < total).astype(jnp.float32)

    sel_row = jnp.sum(_eye_f32(n) * jnp.broadcast_to(sel_col, (n, n)),
                      axis=0, keepdims=True)
    ii = jax.lax.broadcasted_iota(jnp.int32, (n, n), 0)
    jj = jax.lax.broadcasted_iota(jnp.int32, (n, n), 1)
    p_col = jnp.sum(jnp.broadcast_to(sel_row, (n, n)) * (jj < ii), axis=1,
                    keepdims=True)

    tt = jax.lax.broadcasted_iota(jnp.int32, (n, total), 1).astype(jnp.float32)
    b59 = (jnp.broadcast_to(p_col, (n, total)) == tt).astype(jnp.float32)
    b59 = b59 * jnp.broadcast_to(sel_col, (n, total))
    ii_f = jax.lax.broadcasted_iota(jnp.int32, (n, total), 0).astype(jnp.float32)
    tsi_row = jnp.sum(b59 * ii_f, axis=0, keepdims=True)
    ss_col_b = jnp.broadcast_to(_row2col(ss_flat), (n, total))
    tokens_row = jnp.sum(b59 * ss_col_b, axis=0, keepdims=True)

    z59 = (jnp.broadcast_to(rank_col, (n, total)) == tt).astype(jnp.float32)
    s_col_b = jnp.broadcast_to(_row2col(s_flat), (n, total))
    tsp_ref[...] = jnp.sum(z59 * s_col_b, axis=0, keepdims=True)

    g_row = _div10(tsi_row.astype(jnp.int32))
    gp = jnp.broadcast_to(g_row, (nrows, total))
    pp = jax.lax.broadcasted_iota(jnp.int32, (nrows, total), 0)
    oh = (pp == gp).astype(jnp.float32)
    par_col_b = jnp.broadcast_to(_row2col(parents_row), (nrows, total))
    dp_row = jnp.sum(oh * par_col_b, axis=0, keepdims=True)

    tsi_col_b = jnp.broadcast_to(_row2col(tsi_row), (total, total))
    dp_b = jnp.broadcast_to(dp_row, (total, total))
    cnt = jnp.sum((tsi_col_b < dp_b - 1.0).astype(jnp.float32), axis=0,
                  keepdims=True)
    mask_index = jnp.where(dp_row == 0.0, -1.0, cnt) + 1.0
    mi_row = jnp.clip(mask_index, 0.0, float(total))

    nt = total + 1
    rr = jax.lax.broadcasted_iota(jnp.int32, (nt, nt), 0)
    cc = jax.lax.broadcasted_iota(jnp.int32, (nt, nt), 1)
    mi_col = _row2col(mi_row)
    mfull = jnp.concatenate([jnp.zeros((1, 1), jnp.float32), mi_col], axis=0)
    mb = jnp.broadcast_to(mfull, (nt, nt))
    oh = cc.astype(jnp.float32) == mb
    rr_f = rr.astype(jnp.float32)
    isfwd = (mb > rr_f - 1.0) & (rr >= 1)
    base = ((rr == cc) | (cc == 0) | (oh & isfwd)).astype(jnp.float32)
    adj = (oh & (~isfwd) & (rr >= 1)).astype(jnp.float32)
    reach = jnp.maximum(adj, ((rr == cc)).astype(jnp.float32))
    for _ in range(6):
        sq = jax.lax.dot_general(reach, reach, (((1,), (0,)), ((), ())),
                                 preferred_element_type=jnp.float32)
        reach = (reach + sq > 0.0).astype(jnp.float32)
    fin = jax.lax.dot_general(reach, base, (((1,), (0,)), ((), ())),
                              preferred_element_type=jnp.float32)
    tm = (fin > 0.0).astype(jnp.float32)
    tm_ref[...] = tm

    tpi_ref[...] = (jnp.sum(tm, axis=1, keepdims=True) - 1.0).astype(jnp.int32)
    tok_col = _row2col(tokens_row)
    dt = jnp.concatenate(
        [tok_ref[...].astype(jnp.float32), tok_col], axis=0)
    dt_ref[...] = dt.astype(jnp.int32)


def kernel(last_logits, branch_logits, sample_token, total_tokens, depth, top_k):
    d, k, vocab = branch_logits.shape
    total = _TOTAL_TOKENS
    nrows = 1 + d * k
    nbranch = d * k
    rblk = 8

    seg = 100
    seglen = vocab // seg
    nbpad = -(-nbranch // rblk) * rblk
    nsteps = nbpad // rblk

    nt = total + 1
    dt, tsp, tm, tpi = pl.pallas_call(
        functools.partial(_fused_kernel, k=k, d=d, total=total,
                          nsteps=nsteps),
        grid=(nsteps + 1,),
        in_specs=[
            pl.BlockSpec((rblk, seg, seglen),
                         lambda i: (jnp.minimum(i, nsteps - 1), 0, 0)),
            pl.BlockSpec((1, seg, seglen), lambda i: (0, 0, 0)),
            pl.BlockSpec((1, 1), lambda i: (0, 0)),
        ],
        out_specs=[
            pl.BlockSpec((nt, 1), lambda i: (0, 0)),
            pl.BlockSpec((1, total), lambda i: (0, 0)),
            pl.BlockSpec((nt, nt), lambda i: (0, 0)),
            pl.BlockSpec((nt, 1), lambda i: (0, 0)),
        ],
        out_shape=[
            jax.ShapeDtypeStruct((nt, 1), jnp.int32),
            jax.ShapeDtypeStruct((1, total), jnp.float32),
            jax.ShapeDtypeStruct((nt, nt), jnp.float32),
            jax.ShapeDtypeStruct((nt, 1), jnp.int32),
        ],
        scratch_shapes=[
            pltpu.VMEM((nbpad, k), jnp.float32),
            pltpu.VMEM((nbpad, k), jnp.float32),
            pltpu.VMEM((nbpad, 1), jnp.float32),
        ],
    )(branch_logits.reshape(nbranch, seg, seglen),
      last_logits.reshape(1, seg, seglen), sample_token)

    return (
        dt.reshape(1, nt),
        tsp.reshape(total),
        tm.reshape(1, 1, nt, nt),
        tpi.reshape(nt),
    )
